# Initial kernel scaffold; baseline (speedup 1.0000x reference)
import jax
import jax.numpy as jnp
from jax.experimental import pallas as pl


def kernel(V_chanel, mu, W1, b1, W2, b2, W3, b3, W4, b4, W5, b5):
    raise NotImplementedError("write your pallas kernel here")


# speedup vs baseline: 1.1553x; 1.1553x over previous
import jax
import jax.numpy as jnp
from jax import lax
from jax.experimental import pallas as pl
from jax.experimental.pallas import tpu as pltpu
from jax.experimental.pallas import tpu_sc as plsc

B = 16
NPIX = 512 * 512
HALF = NPIX // 2
CH = 32768
NCH = HALF // CH
NBINS = 256
NUM_ITERS = 8
L = 16


def _sc_hist_body(v_hbm, hist_hbm, mm_hbm,
                  buf, subhist, hh, tmp, mmb, mmo, wbuf, sh_mm, sh_h):
    c = lax.axis_index("c")
    s = lax.axis_index("s")
    b = c * 8 + s // 2
    half = s % 2
    base = half * HALF

    def chunk_minmax(k, carry):
        mn, mx = carry
        pltpu.sync_copy(v_hbm.at[b, pl.ds(base + k * CH, CH)], buf)

        def body(i, c2):
            m1, m2 = c2
            v = buf[pl.ds(i * L, L)]
            return jnp.minimum(m1, v), jnp.maximum(m2, v)

        return lax.fori_loop(0, CH // L, body, (mn, mx))

    big = jnp.full((L,), jnp.inf, jnp.float32)
    mnv, mxv = lax.fori_loop(0, NCH, chunk_minmax, (big, -big))

    mmb[pl.ds(0, L)] = mnv
    mmb[pl.ds(L, L)] = mxv
    pltpu.sync_copy(mmb, sh_mm.at[s])
    plsc.subcore_barrier()
    pltpu.sync_copy(sh_mm.at[s ^ 1], mmo)

    mv = jnp.minimum(mnv, mmo[pl.ds(0, L)])
    xv = jnp.maximum(mxv, mmo[pl.ds(L, L)])
    mn_all = mv[0]
    mx_all = xv[0]
    for j in range(1, L):
        mn_all = jnp.minimum(mn_all, mv[j])
        mx_all = jnp.maximum(mx_all, xv[j])

    rng = mx_all - mn_all
    safe = jnp.where(rng == 0.0, 1.0, rng)
    safe_v = jnp.full((L,), 1.0, jnp.float32) * safe
    a_v = jnp.full((L,), float(NBINS), jnp.float32) / safe_v
    b_v = jnp.full((L,), 1.0, jnp.float32) * (-mn_all) * a_v
    lane = lax.iota(jnp.int32, L) * NBINS
    ones = jnp.full((L,), 1.0, jnp.float32)

    def zbody(i, _):
        subhist[pl.ds(i * L, L)] = jnp.zeros((L,), jnp.float32)
        return 0

    lax.fori_loop(0, (NBINS * L) // L, zbody, 0)

    def chunk_hist(k, _):
        pltpu.sync_copy(v_hbm.at[b, pl.ds(base + k * CH, CH)], buf)

        def body(i, _2):
            v = buf[pl.ds(i * L, L)]
            t = v * a_v + b_v
            ti = t.astype(jnp.int32)
            ti = jnp.clip(ti, 0, NBINS - 1) + lane
            plsc.addupdate_scatter(subhist, [ti], ones)
            return 0

        lax.fori_loop(0, CH // L, body, 0)
        return 0

    lax.fori_loop(0, NCH, chunk_hist, 0)

    for g in range(NBINS // L):
        acc = jnp.zeros((L,), jnp.float32)
        for l in range(L):
            acc = acc + subhist[pl.ds(l * NBINS + g * L, L)]
        hh[pl.ds(g * L, L)] = acc

    pltpu.sync_copy(hh, sh_h.at[s])
    plsc.subcore_barrier()

    @pl.when(half == 0)
    def _():
        pltpu.sync_copy(sh_h.at[s + 1], tmp)
        inv = jnp.float32(1.0 / NPIX)
        for g in range(NBINS // L):
            tot = (hh[pl.ds(g * L, L)] + tmp[pl.ds(g * L, L)]) * inv
            hh[pl.ds(g * L, L)] = tot
        pltpu.sync_copy(hh, hist_hbm.at[b])
        iota16 = lax.iota(jnp.int32, L)
        wbuf[...] = jnp.where(iota16 == 0, mn_all, mx_all)
        pltpu.sync_copy(wbuf, mm_hbm.at[b])


def _sc_hist(v2):
    f = pl.kernel(
        _sc_hist_body,
        out_type=[jax.ShapeDtypeStruct((B, NBINS), jnp.float32),
                  jax.ShapeDtypeStruct((B, L), jnp.float32)],
        mesh=plsc.VectorSubcoreMesh(core_axis_name="c", subcore_axis_name="s"),
        compiler_params=pltpu.CompilerParams(needs_layout_passes=False),
        scratch_types=[
            pltpu.VMEM((CH,), jnp.float32),
            pltpu.VMEM((NBINS * L,), jnp.float32),
            pltpu.VMEM((NBINS,), jnp.float32),
            pltpu.VMEM((NBINS,), jnp.float32),
            pltpu.VMEM((2 * L,), jnp.float32),
            pltpu.VMEM((2 * L,), jnp.float32),
            pltpu.VMEM((L,), jnp.float32),
            pltpu.VMEM_SHARED((L, 2 * L), jnp.float32),
            pltpu.VMEM_SHARED((L, NBINS), jnp.float32),
        ],
    )
    return f(v2)


def _lrelu(x):
    return jnp.where(x >= 0, x, 0.01 * x)


def _mlp_body(hist_ref, mm_ref, mu_ref,
              w1h_ref, w1t_ref, b1_ref, w2_ref, b2_ref,
              w3a_ref, w3h_ref, w3t_ref, b3_ref,
              w4_ref, b4_ref, w5_ref, b5_ref, al_ref):
    h = hist_ref[...]
    mn = mm_ref[:, 0:1]
    mx = mm_ref[:, 1:2]
    mu = mu_ref[...]

    def tail(wt_ref):
        return (mn * wt_ref[0:1, :] + mx * wt_ref[1:2, :]
                + mu * wt_ref[2:3, :])

    x = _lrelu(jnp.dot(h, w1h_ref[...]) + tail(w1t_ref) + b1_ref[...])
    x = _lrelu(jnp.dot(x, w2_ref[...]) + b2_ref[...])
    x = _lrelu(jnp.dot(x, w3a_ref[...]) + jnp.dot(h, w3h_ref[...])
               + tail(w3t_ref) + b3_ref[...])
    x = _lrelu(jnp.dot(x, w4_ref[...]) + b4_ref[...])
    al_ref[...] = _lrelu(jnp.dot(x, w5_ref[...]) + b5_ref[...])


def _mlp(hist, mm, mu, W1, b1, W2, b2, W3, b3, W4, b4, W5, b5):
    args = (hist, mm, mu,
            W1[:NBINS], W1[NBINS:], b1[None, :], W2, b2[None, :],
            W3[:64], W3[64:64 + NBINS], W3[64 + NBINS:], b3[None, :],
            W4, b4[None, :], W5, b5[None, :])
    return pl.pallas_call(
        _mlp_body,
        out_shape=jax.ShapeDtypeStruct((B, NUM_ITERS), jnp.float32),
        in_specs=[pl.BlockSpec(memory_space=pltpu.VMEM)] * len(args),
        out_specs=pl.BlockSpec(memory_space=pltpu.VMEM),
    )(*args)


PIX_R = 128
PIX_C = 2048


def _pix_body(al_ref, v_ref, o_ref):
    b = pl.program_id(0)
    x = v_ref[0]
    for i in range(NUM_ITERS):
        a = al_ref[b, i]
        x = x + a * (x - x * x)
    o_ref[0] = x


def _pix_update(v3, alphas):
    nblk = NPIX // (PIX_R * PIX_C)
    return pl.pallas_call(
        _pix_body,
        grid=(B, nblk),
        in_specs=[pl.BlockSpec(memory_space=pltpu.SMEM),
                  pl.BlockSpec((1, PIX_R, PIX_C), lambda b, k: (b, k, 0))],
        out_specs=pl.BlockSpec((1, PIX_R, PIX_C), lambda b, k: (b, k, 0)),
        out_shape=jax.ShapeDtypeStruct((B, PIX_R * nblk, PIX_C), jnp.float32),
    )(alphas, v3)


def kernel(V_chanel, mu, W1, b1, W2, b2, W3, b3, W4, b4, W5, b5):
    v2 = V_chanel.reshape(B, NPIX)
    hist, mm = _sc_hist(v2)
    alphas = _mlp(hist, mm, mu, W1, b1, W2, b2, W3, b3, W4, b4, W5, b5)
    nblk = NPIX // (PIX_R * PIX_C)
    v3 = V_chanel.reshape(B, PIX_R * nblk, PIX_C)
    out = _pix_update(v3, alphas)
    return out.reshape(B, 1, 512, 512)


# speedup vs baseline: 3.7941x; 3.2842x over previous
import jax
import jax.numpy as jnp
from jax import lax
from jax.experimental import pallas as pl
from jax.experimental.pallas import tpu as pltpu
from jax.experimental.pallas import tpu_sc as plsc

B = 16
H = 512
W = 512
NPIX = H * W
CHR = 64
CH = CHR * W
NCH = (NPIX // 2) // CH
NBINS = 256
NUM_ITERS = 8
L = 16
NVREG = CH // L
VPR = W // L


def _sc_hist_body(v_hbm, hist_hbm, mm_hbm,
                  buf0, buf1, subhist, hh, tmp, mmb, mmo, wbuf,
                  sh_mm, sh_h, sem0, sem1):
    c = lax.axis_index("c")
    s = lax.axis_index("s")
    b = c * 8 + s // 2
    half = s % 2
    row_base = half * (H // 2)

    bufs = (buf0, buf1)
    sems = (sem0, sem1)

    def chunk_copy(k):
        src = v_hbm.at[b, 0, pl.ds(row_base + k * CHR, CHR), :]
        return pltpu.make_async_copy(src, bufs[k % 2], sems[k % 2])

    cpa = [chunk_copy(k) for k in range(NCH)]
    cpa[0].start()
    big = jnp.full((L,), jnp.inf, jnp.float32)
    accs = (big, -big, big, -big, big, -big, big, -big)
    for k in range(NCH):
        if k + 1 < NCH:
            cpa[k + 1].start()
        cpa[k].wait()
        cur = bufs[k % 2]

        @plsc.parallel_loop(0, NVREG, step=4, unroll=2, carry=accs)
        def _mmloop(i, carry, cur=cur):
            out = []
            for u in range(4):
                j = i + u
                v = cur[j // VPR, pl.ds((j % VPR) * L, L)]
                out.append(jnp.minimum(carry[2 * u], v))
                out.append(jnp.maximum(carry[2 * u + 1], v))
            return tuple(out)

        accs = _mmloop

    mnv = jnp.minimum(jnp.minimum(accs[0], accs[2]),
                      jnp.minimum(accs[4], accs[6]))
    mxv = jnp.maximum(jnp.maximum(accs[1], accs[3]),
                      jnp.maximum(accs[5], accs[7]))

    cpb = [chunk_copy(k) for k in range(NCH)]
    cpb[0].start()

    mmb[pl.ds(0, L)] = mnv
    mmb[pl.ds(L, L)] = mxv
    pltpu.sync_copy(mmb, sh_mm.at[s])
    plsc.subcore_barrier()
    pltpu.sync_copy(sh_mm.at[s ^ 1], mmo)

    mv = jnp.minimum(mnv, mmo[pl.ds(0, L)])
    xv = jnp.maximum(mxv, mmo[pl.ds(L, L)])
    mn_all = mv[0]
    mx_all = xv[0]
    for j in range(1, L):
        mn_all = jnp.minimum(mn_all, mv[j])
        mx_all = jnp.maximum(mx_all, xv[j])

    rng = mx_all - mn_all
    safe = jnp.where(rng == 0.0, 1.0, rng)
    safe_v = jnp.full((L,), 1.0, jnp.float32) * safe
    a_v = jnp.full((L,), float(NBINS), jnp.float32) / safe_v
    lane_f = lax.iota(jnp.int32, L).astype(jnp.float32) * float(NBINS)
    b_v = lane_f - (jnp.full((L,), 1.0, jnp.float32) * mn_all) * a_v
    lo_v = lane_f
    hi_v = lane_f + float(NBINS - 1)
    ones = jnp.full((L,), 1.0, jnp.float32)

    @plsc.parallel_loop(0, (NBINS * L) // L, step=1, unroll=4)
    def _zloop(i):
        subhist[pl.ds(i * L, L)] = jnp.zeros((L,), jnp.float32)

    cpb[1].start()

    for k in range(NCH):
        cpb[k].wait()
        cur = bufs[k % 2]

        @plsc.parallel_loop(0, NVREG, step=4, unroll=2)
        def _hloop(i, cur=cur):
            for u in range(4):
                j = i + u
                v = cur[j // VPR, pl.ds((j % VPR) * L, L)]
                t = v * a_v + b_v
                t = jnp.minimum(jnp.maximum(t, lo_v), hi_v)
                ti = t.astype(jnp.int32)
                plsc.addupdate_scatter(subhist, [ti], ones)

        if k + 2 < NCH:
            cpb[k + 2].start()

    for g in range(NBINS // L):
        acc = jnp.zeros((L,), jnp.float32)
        for l in range(L):
            acc = acc + subhist[pl.ds(l * NBINS + g * L, L)]
        hh[pl.ds(g * L, L)] = acc

    pltpu.sync_copy(hh, sh_h.at[s])
    plsc.subcore_barrier()

    @pl.when(half == 0)
    def _():
        pltpu.sync_copy(sh_h.at[s + 1], tmp)
        inv = jnp.float32(1.0 / NPIX)
        for g in range(NBINS // L):
            tot = (hh[pl.ds(g * L, L)] + tmp[pl.ds(g * L, L)]) * inv
            hh[pl.ds(g * L, L)] = tot
        pltpu.sync_copy(hh, hist_hbm.at[b])
        iota16 = lax.iota(jnp.int32, L)
        wbuf[...] = jnp.where(iota16 == 0, mn_all, mx_all)
        pltpu.sync_copy(wbuf, mm_hbm.at[b])


def _sc_hist(v4):
    f = pl.kernel(
        _sc_hist_body,
        out_type=[jax.ShapeDtypeStruct((B, NBINS), jnp.float32),
                  jax.ShapeDtypeStruct((B, L), jnp.float32)],
        mesh=plsc.VectorSubcoreMesh(core_axis_name="c", subcore_axis_name="s"),
        compiler_params=pltpu.CompilerParams(needs_layout_passes=False),
        scratch_types=[
            pltpu.VMEM((CHR, W), jnp.float32),
            pltpu.VMEM((CHR, W), jnp.float32),
            pltpu.VMEM((NBINS * L,), jnp.float32),
            pltpu.VMEM((NBINS,), jnp.float32),
            pltpu.VMEM((NBINS,), jnp.float32),
            pltpu.VMEM((2 * L,), jnp.float32),
            pltpu.VMEM((2 * L,), jnp.float32),
            pltpu.VMEM((L,), jnp.float32),
            pltpu.VMEM_SHARED((L, 2 * L), jnp.float32),
            pltpu.VMEM_SHARED((L, NBINS), jnp.float32),
            pltpu.SemaphoreType.DMA,
            pltpu.SemaphoreType.DMA,
        ],
    )
    return f(v4)


def _lrelu(x):
    return jnp.where(x >= 0, x, 0.01 * x)


def _mlp_body(hist_ref, mm_ref, mu_ref,
              w1h_ref, w1t_ref, b1_ref, w2_ref, b2_ref,
              w3a_ref, w3h_ref, w3t_ref, b3_ref,
              w4_ref, b4_ref, w5_ref, b5_ref, al_ref):
    h = hist_ref[...]
    mn = mm_ref[:, 0:1]
    mx = mm_ref[:, 1:2]
    mu = mu_ref[...]

    def tail(wt_ref):
        return (mn * wt_ref[0:1, :] + mx * wt_ref[1:2, :]
                + mu * wt_ref[2:3, :])

    x = _lrelu(jnp.dot(h, w1h_ref[...]) + tail(w1t_ref) + b1_ref[...])
    x = _lrelu(jnp.dot(x, w2_ref[...]) + b2_ref[...])
    x = _lrelu(jnp.dot(x, w3a_ref[...]) + jnp.dot(h, w3h_ref[...])
               + tail(w3t_ref) + b3_ref[...])
    x = _lrelu(jnp.dot(x, w4_ref[...]) + b4_ref[...])
    al_ref[...] = _lrelu(jnp.dot(x, w5_ref[...]) + b5_ref[...])


def _mlp(hist, mm, mu, W1, b1, W2, b2, W3, b3, W4, b4, W5, b5):
    args = (hist, mm, mu,
            W1[:NBINS], W1[NBINS:], b1[None, :], W2, b2[None, :],
            W3[:64], W3[64:64 + NBINS], W3[64 + NBINS:], b3[None, :],
            W4, b4[None, :], W5, b5[None, :])
    return pl.pallas_call(
        _mlp_body,
        out_shape=jax.ShapeDtypeStruct((B, NUM_ITERS), jnp.float32),
        in_specs=[pl.BlockSpec(memory_space=pltpu.VMEM)] * len(args),
        out_specs=pl.BlockSpec(memory_space=pltpu.VMEM),
    )(*args)


def _pix_body(al_ref, v_ref, o_ref):
    b = pl.program_id(0)
    x = v_ref[0, 0]
    for i in range(NUM_ITERS):
        a = al_ref[b, i]
        x = x * ((1.0 + a) - a * x)
    o_ref[0, 0] = x


def _pix_update(v4, alphas):
    return pl.pallas_call(
        _pix_body,
        grid=(B,),
        in_specs=[pl.BlockSpec(memory_space=pltpu.SMEM),
                  pl.BlockSpec((1, 1, H, W), lambda b: (b, 0, 0, 0))],
        out_specs=pl.BlockSpec((1, 1, H, W), lambda b: (b, 0, 0, 0)),
        out_shape=jax.ShapeDtypeStruct((B, 1, H, W), jnp.float32),
    )(alphas, v4)


def kernel(V_chanel, mu, W1, b1, W2, b2, W3, b3, W4, b4, W5, b5):
    hist, mm = _sc_hist(V_chanel)
    alphas = _mlp(hist, mm, mu, W1, b1, W2, b2, W3, b3, W4, b4, W5, b5)
    return _pix_update(V_chanel, alphas)
